# baseline (speedup 1.0000x reference)
import functools

import jax
import jax.numpy as jnp
from jax import lax
from jax.experimental import pallas as pl
from jax.experimental.pallas import tpu as pltpu
from jax.experimental.pallas import tpu_sc as plsc

_GCHUNK = 512
_IDXBLK = 2 * _GCHUNK


def _make_gather(V, D, N):
    info = plsc.get_sparse_core_info()
    NC, NS = info.num_cores, info.num_subcores
    NW = NC * NS
    assert N % (NW * _IDXBLK) == 0
    n_per_w = N // NW
    n_it = n_per_w // _IDXBLK

    mesh = plsc.VectorSubcoreMesh(core_axis_name="c", subcore_axis_name="s")

    @functools.partial(
        pl.kernel,
        out_type=jax.ShapeDtypeStruct((N, D), jnp.float32),
        mesh=mesh,
        scratch_types=[
            pltpu.VMEM((_IDXBLK,), jnp.int32),
            pltpu.VMEM((_GCHUNK, D), jnp.float32),
            pltpu.VMEM((_GCHUNK, D), jnp.float32),
            pltpu.SemaphoreType.DMA,
            pltpu.SemaphoreType.DMA,
        ],
        compiler_params=pltpu.CompilerParams(use_tc_tiling_on_sc=False),
    )
    def gather_kernel(w_hbm, idx_hbm, out_hbm, idx_buf, rows0, rows1, gsem, wsem):
        wid = lax.axis_index("s") * NC + lax.axis_index("c")
        wbase = wid * n_per_w
        bufs = (rows0, rows1)

        def body(i, carry):
            base = pl.multiple_of(wbase + i * _IDXBLK, _IDXBLK)
            pltpu.sync_copy(idx_hbm.at[pl.ds(base, _IDXBLK)], idx_buf)
            for s in range(2):
                buf = bufs[s]
                @pl.when(i > 0)
                def _():
                    pltpu.make_async_copy(
                        buf, out_hbm.at[pl.ds(0, _GCHUNK)], wsem
                    ).wait()
                pltpu.async_copy(
                    w_hbm.at[idx_buf.at[pl.ds(s * _GCHUNK, _GCHUNK)]],
                    buf,
                    gsem,
                ).wait()
                pltpu.async_copy(
                    buf, out_hbm.at[pl.ds(base + s * _GCHUNK, _GCHUNK)], wsem
                )
            return carry

        lax.fori_loop(0, n_it, body, 0)
        for s in range(2):
            pltpu.make_async_copy(
                bufs[s], out_hbm.at[pl.ds(0, _GCHUNK)], wsem
            ).wait()

    return gather_kernel


def kernel(q, W):
    B, L = q.shape
    V, D = W.shape
    N = B * L
    idx = q.reshape(N).astype(jnp.int32)
    out = _make_gather(V, D, N)(W, idx)
    return out.reshape(B, L, D)
